# baseline (speedup 1.0000x reference)
import functools

import jax
import jax.numpy as jnp
from jax import lax
from jax.experimental import pallas as pl
from jax.experimental.pallas import tpu as pltpu
from jax.experimental.pallas import tpu_sc as plsc

B, N, FIN, FOUT, E = 4, 10000, 128, 128, 320000
KP1 = 4
NSC = 2
NT = 16
BPS = B // NSC
CH = 128
SCH = 16
EPT = 20480
E_PAD = EPT * NT
NCHUNK = EPT // CH
NSUPER = NCHUNK // SCH
NP = 10240
RPT = NP // NT
RJ = 128

BN = 400

_BCAST_DNUMS = lax.GatherDimensionNumbers(
    offset_dims=(), collapsed_slice_dims=(0,), start_index_map=(0,))


def _bcast_lane(vec, j):
    return lax.gather(
        vec, jnp.full((16, 1), j, jnp.int32), _BCAST_DNUMS, slice_sizes=(1,),
        mode=lax.GatherScatterMode.PROMISE_IN_BOUNDS)


def _mm_body(x_ref, w_ref, o_ref):
    o_ref[0] = lax.dot_general(
        x_ref[0], w_ref[0], (((1,), (0,)), ((), ())),
        preferred_element_type=jnp.float32)


def _tc_xw(x, weight):
    return pl.pallas_call(
        _mm_body,
        grid=(KP1 * B, N // BN),
        in_specs=[
            pl.BlockSpec((1, BN, FIN), lambda i, j: (i % B, j, 0)),
            pl.BlockSpec((1, FIN, FOUT), lambda i, j: (i // B, 0, 0)),
        ],
        out_specs=pl.BlockSpec((1, BN, FOUT), lambda i, j: (i, j, 0)),
        out_shape=jax.ShapeDtypeStruct((KP1 * B, N, FOUT), jnp.float32),
    )(x, weight)


@functools.partial(
    pl.kernel,
    out_type=jax.ShapeDtypeStruct((B, NP, FOUT), jnp.float32),
    mesh=plsc.VectorSubcoreMesh(core_axis_name="c", subcore_axis_name="s"),
    scratch_types=[
        pltpu.VMEM_SHARED((NP, FOUT), jnp.float32),
        pltpu.VMEM((SCH, CH), jnp.int32),
        pltpu.VMEM((SCH, CH), jnp.int32),
        pltpu.VMEM((SCH, CH), jnp.float32),
        pltpu.VMEM((CH, FOUT), jnp.float32),
        pltpu.VMEM((CH, FOUT), jnp.float32),
        pltpu.SemaphoreType.DMA,
        pltpu.SemaphoreType.DMA,
        pltpu.SemaphoreType.DMA,
        pltpu.SemaphoreType.DMA,
    ],
)
def _sc_spmm(xw_hbm, col_hbm, row_hbm, val_hbm, binit_hbm, out_hbm,
             acc, fcol, rowv, valv, gbuf0, gbuf1, sem0, sem1, sem2, sem3):
    cid = lax.axis_index("c")
    sid = lax.axis_index("s")

    def _scale(gb, c):
        def group_body(eg, _):
            vvec = valv[c, pl.ds(eg * 16, 16)]
            for j in range(16):
                vb = _bcast_lane(vvec, j)
                i = eg * 16 + j
                for g in range(FOUT // 16):
                    gb[i, pl.ds(g * 16, 16)] = gb[i, pl.ds(g * 16, 16)] * vb
            return 0

        lax.fori_loop(0, CH // 16, group_body, 0)

    for bl in range(BPS):
        b = cid * BPS + bl
        pltpu.sync_copy(binit_hbm, gbuf0)
        for j in range(RPT // RJ):
            pltpu.sync_copy(gbuf0, acc.at[pl.ds(sid * RPT + j * RJ, RJ)])
        plsc.subcore_barrier()
        for k in range(KP1):
            offv = jnp.full((16,), k * B * N, jnp.int32) + b * N

            def super_body(s, _, k=k, offv=offv):
                pltpu.sync_copy(col_hbm.at[k, sid, pl.ds(s * SCH, SCH)], fcol)
                pltpu.sync_copy(row_hbm.at[k, sid, pl.ds(s * SCH, SCH)], rowv)
                pltpu.sync_copy(val_hbm.at[k, sid, pl.ds(s * SCH, SCH)], valv)

                def add_off(r, _):
                    for g in range(CH // 16):
                        sl = pl.ds(g * 16, 16)
                        fcol[r, sl] = fcol[r, sl] + offv
                    return 0

                lax.fori_loop(0, SCH, add_off, 0)
                pltpu.async_copy(xw_hbm.at[fcol.at[0]], gbuf0, sem0)

                def pair_body(p, _):
                    c0 = 2 * p
                    c1 = c0 + 1

                    @pl.when(p > 0)
                    def _():
                        pltpu.make_async_copy(
                            gbuf1, acc.at[rowv.at[c1]], sem3).wait()

                    pltpu.async_copy(xw_hbm.at[fcol.at[c1]], gbuf1, sem1)
                    pltpu.make_async_copy(
                        xw_hbm.at[fcol.at[c0]], gbuf0, sem0).wait()
                    _scale(gbuf0, c0)
                    pltpu.async_copy(
                        gbuf0, acc.at[rowv.at[c0]], sem2, add=True)
                    pltpu.make_async_copy(
                        xw_hbm.at[fcol.at[c1]], gbuf1, sem1).wait()
                    _scale(gbuf1, c1)
                    pltpu.make_async_copy(
                        gbuf0, acc.at[rowv.at[c0]], sem2).wait()

                    @pl.when(p < SCH // 2 - 1)
                    def _():
                        pltpu.async_copy(
                            xw_hbm.at[fcol.at[c0 + 2]], gbuf0, sem0)

                    pltpu.async_copy(
                        gbuf1, acc.at[rowv.at[c1]], sem3, add=True)
                    return 0

                lax.fori_loop(0, SCH // 2, pair_body, 0)
                pltpu.make_async_copy(
                    gbuf1, acc.at[rowv.at[SCH - 1]], sem3).wait()
                return 0

            lax.fori_loop(0, NSUPER, super_body, 0)
        plsc.subcore_barrier()
        for j in range(RPT // RJ):
            rs = sid * RPT + j * RJ
            pltpu.sync_copy(acc.at[pl.ds(rs, RJ)], out_hbm.at[b, pl.ds(rs, RJ)])
        plsc.subcore_barrier()


def kernel(x, cheb_indices, cheb_values, weight, bias):
    xw = _tc_xw(x, weight).reshape(KP1 * B * N, FOUT)
    pad = E_PAD - E
    col = jnp.pad(cheb_indices[:, 1, :], ((0, 0), (0, pad)))
    row = jnp.pad(cheb_indices[:, 0, :], ((0, 0), (0, pad)))
    val = jnp.pad(cheb_values, ((0, 0), (0, pad)))
    col = col.reshape(KP1, NT, NCHUNK, CH)
    row = row.reshape(KP1, NT, NCHUNK, CH)
    val = val.reshape(KP1, NT, NCHUNK, CH)
    binit = jnp.broadcast_to(bias[None, :], (RJ, FOUT))
    return _sc_spmm(xw, col, row, val, binit)[:, :N, :]


# speedup vs baseline: 2.2319x; 2.2319x over previous
import functools

import jax
import jax.numpy as jnp
from jax import lax
from jax.experimental import pallas as pl
from jax.experimental.pallas import tpu as pltpu
from jax.experimental.pallas import tpu_sc as plsc

B, N, FIN, FOUT, E = 4, 10000, 128, 128, 320000
KP1 = 4
NSC = 2
NT = 16
BPS = B // NSC
CH = 128
SCH = 16
EPT = 20480
E_PAD = EPT * NT
NCHUNK = EPT // CH
NSUPER = NCHUNK // SCH
NP = 10240
RPT = NP // NT
RJ = 128

BN = 400

_BCAST_DNUMS = lax.GatherDimensionNumbers(
    offset_dims=(), collapsed_slice_dims=(0,), start_index_map=(0,))


def _bcast_lane(vec, j):
    return lax.gather(
        vec, jnp.full((16, 1), j, jnp.int32), _BCAST_DNUMS, slice_sizes=(1,),
        mode=lax.GatherScatterMode.PROMISE_IN_BOUNDS)


def _mm_body(x_ref, w_ref, o_ref):
    o_ref[0] = lax.dot_general(
        x_ref[0], w_ref[0], (((1,), (0,)), ((), ())),
        preferred_element_type=jnp.float32)


def _tc_xw(x, weight):
    return pl.pallas_call(
        _mm_body,
        grid=(KP1 * B, N // BN),
        in_specs=[
            pl.BlockSpec((1, BN, FIN), lambda i, j: (i % B, j, 0)),
            pl.BlockSpec((1, FIN, FOUT), lambda i, j: (i // B, 0, 0)),
        ],
        out_specs=pl.BlockSpec((1, BN, FOUT), lambda i, j: (i, j, 0)),
        out_shape=jax.ShapeDtypeStruct((KP1 * B, N, FOUT), jnp.float32),
    )(x, weight)


@functools.partial(
    pl.kernel,
    out_type=jax.ShapeDtypeStruct((B, NP, FOUT), jnp.float32),
    mesh=plsc.VectorSubcoreMesh(core_axis_name="c", subcore_axis_name="s"),
    scratch_types=[
        pltpu.VMEM_SHARED((NP, FOUT), jnp.float32),
        pltpu.VMEM((SCH, CH), jnp.int32),
        pltpu.VMEM((SCH, CH), jnp.int32),
        pltpu.VMEM((SCH, CH), jnp.float32),
        pltpu.VMEM((CH, FOUT), jnp.float32),
        pltpu.VMEM((CH, FOUT), jnp.float32),
        pltpu.SemaphoreType.DMA,
        pltpu.SemaphoreType.DMA,
    ],
)
def _sc_spmm(xw_hbm, col_hbm, row_hbm, val_hbm, binit_hbm, out_hbm,
             acc, fcol, rowv, valv, gbuf0, gbuf1, sem0, sem1):
    cid = lax.axis_index("c")
    sid = lax.axis_index("s")

    def _scale(gb, c):
        def group_body(eg, _):
            vvec = valv[c, pl.ds(eg * 16, 16)]
            for j in range(16):
                vb = _bcast_lane(vvec, j)
                i = eg * 16 + j
                for g in range(FOUT // 16):
                    gb[i, pl.ds(g * 16, 16)] = gb[i, pl.ds(g * 16, 16)] * vb
            return 0

        lax.fori_loop(0, CH // 16, group_body, 0)

    for bl in range(BPS):
        b = cid * BPS + bl
        pltpu.sync_copy(binit_hbm, gbuf0)
        for j in range(RPT // RJ):
            pltpu.sync_copy(gbuf0, acc.at[pl.ds(sid * RPT + j * RJ, RJ)])
        plsc.subcore_barrier()
        for k in range(KP1):
            offv = jnp.full((16,), k * B * N, jnp.int32) + b * N

            def super_body(s, _, k=k, offv=offv):
                pltpu.sync_copy(col_hbm.at[k, sid, pl.ds(s * SCH, SCH)], fcol)
                pltpu.sync_copy(row_hbm.at[k, sid, pl.ds(s * SCH, SCH)], rowv)
                pltpu.sync_copy(val_hbm.at[k, sid, pl.ds(s * SCH, SCH)], valv)

                def add_off(r, _):
                    for g in range(CH // 16):
                        sl = pl.ds(g * 16, 16)
                        fcol[r, sl] = fcol[r, sl] + offv
                    return 0

                lax.fori_loop(0, SCH, add_off, 0)
                pltpu.async_copy(xw_hbm.at[fcol.at[0]], gbuf0, sem0)

                def pair_body(p, _):
                    c0 = 2 * p
                    c1 = c0 + 1
                    pltpu.async_copy(xw_hbm.at[fcol.at[c1]], gbuf1, sem1)
                    pltpu.make_async_copy(
                        xw_hbm.at[fcol.at[c0]], gbuf0, sem0).wait()
                    _scale(gbuf0, c0)
                    pltpu.sync_copy(gbuf0, acc.at[rowv.at[c0]], add=True)

                    @pl.when(p < SCH // 2 - 1)
                    def _():
                        pltpu.async_copy(
                            xw_hbm.at[fcol.at[c0 + 2]], gbuf0, sem0)

                    pltpu.make_async_copy(
                        xw_hbm.at[fcol.at[c1]], gbuf1, sem1).wait()
                    _scale(gbuf1, c1)
                    pltpu.sync_copy(gbuf1, acc.at[rowv.at[c1]], add=True)
                    return 0

                lax.fori_loop(0, SCH // 2, pair_body, 0)
                return 0

            lax.fori_loop(0, NSUPER, super_body, 0)
        plsc.subcore_barrier()
        for j in range(RPT // RJ):
            rs = sid * RPT + j * RJ
            pltpu.sync_copy(acc.at[pl.ds(rs, RJ)], out_hbm.at[b, pl.ds(rs, RJ)])
        plsc.subcore_barrier()


def kernel(x, cheb_indices, cheb_values, weight, bias):
    xw = _tc_xw(x, weight).reshape(KP1 * B * N, FOUT)
    pad = E_PAD - E
    padv = jnp.broadcast_to(
        (jnp.arange(pad, dtype=jnp.int32) * 16) % N, (KP1, pad))
    col = jnp.concatenate([cheb_indices[:, 1, :], padv], axis=1)
    row = jnp.concatenate([cheb_indices[:, 0, :], padv], axis=1)
    val = jnp.pad(cheb_values, ((0, 0), (0, pad)))
    col = col.reshape(KP1, NT, NCHUNK, CH)
    row = row.reshape(KP1, NT, NCHUNK, CH)
    val = val.reshape(KP1, NT, NCHUNK, CH)
    binit = jnp.broadcast_to(bias[None, :], (RJ, FOUT))
    return _sc_spmm(xw, col, row, val, binit)[:, :N, :]
